# Initial kernel scaffold; baseline (speedup 1.0000x reference)
import jax
import jax.numpy as jnp
from jax.experimental import pallas as pl


def kernel(x, edge_index, edge_attr, eW1, eb1, eW2, eb2, uW1, ub1, uW2, ub2, eps):
    raise NotImplementedError("write your pallas kernel here")


# speedup vs baseline: 1.8034x; 1.8034x over previous
import functools
import math

import jax
import jax.numpy as jnp
from jax import lax
from jax.experimental import pallas as pl
from jax.experimental.pallas import tpu as pltpu
from jax.experimental.pallas import tpu_sc as plsc

N_NODES = 10000
N_EDGES = 320000
H = 128

NC = 2
NS = 16
NW = NC * NS
EPW = N_EDGES // NW
CHUNK = 80
NCHUNK = EPW // CHUNK
NT_IO = 10
RPT = N_NODES // NT_IO

_C1 = 2.0 * math.sqrt(2.0 / math.pi)
_C2 = _C1 * 0.044715


def _mm_bias_body(a_ref, w_ref, b_ref, o_ref):
    o_ref[...] = (
        jnp.dot(a_ref[...], w_ref[...],
                preferred_element_type=jnp.float32,
                precision=lax.Precision.HIGHEST)
        + b_ref[...]
    )


def _mm_bias(a, w, b, block_rows):
    rows = a.shape[0]
    grid = rows // block_rows
    return pl.pallas_call(
        _mm_bias_body,
        grid=(grid,),
        in_specs=[
            pl.BlockSpec((block_rows, H), lambda i: (i, 0)),
            pl.BlockSpec((H, H), lambda i: (0, 0)),
            pl.BlockSpec((1, H), lambda i: (0, 0)),
        ],
        out_specs=pl.BlockSpec((block_rows, H), lambda i: (i, 0)),
        out_shape=jax.ShapeDtypeStruct((rows, H), jnp.float32),
    )(a, w, b)


@functools.partial(
    pl.kernel,
    out_type=jax.ShapeDtypeStruct((NC, N_NODES, H), jnp.float32),
    mesh=plsc.VectorSubcoreMesh(core_axis_name="c", subcore_axis_name="s"),
    scratch_types=[
        pltpu.VMEM((CHUNK, H), jnp.float32),
        pltpu.VMEM((CHUNK, H), jnp.float32),
        pltpu.VMEM((1, H), jnp.float32),
        pltpu.VMEM((CHUNK,), jnp.int32),
        pltpu.VMEM((CHUNK,), jnp.int32),
        pltpu.SemaphoreType.DMA,
        pltpu.VMEM_SHARED((N_NODES, H), jnp.float32),
    ],
)
def _sc_edge(t_hbm, xw_hbm, src_hbm, dst_hbm, c_hbm, s_out,
             t_buf, g_buf, c_buf, sidx, didx, sem, s_sh):
    cid = lax.axis_index("c")
    sid = lax.axis_index("s")
    wid = cid * NS + sid

    zero16 = jnp.zeros((16,), jnp.float32)

    @pl.loop(0, CHUNK)
    def _init(e):
        for j in range(H // 16):
            t_buf[e, pl.ds(j * 16, 16)] = zero16

    pltpu.sync_copy(c_hbm, c_buf)

    rbase = sid * RPT
    nfull = RPT // CHUNK
    rem = RPT - nfull * CHUNK

    @pl.when(sid < NT_IO)
    def _zero():
        for k in range(nfull):
            pltpu.sync_copy(t_buf, s_sh.at[pl.ds(rbase + k * CHUNK, CHUNK)])
        pltpu.sync_copy(t_buf.at[pl.ds(0, rem)],
                        s_sh.at[pl.ds(rbase + nfull * CHUNK, rem)])

    plsc.subcore_barrier()

    ebase0 = wid * EPW

    @pl.loop(0, NCHUNK)
    def _chunk(c):
        ebase = ebase0 + c * CHUNK
        pltpu.sync_copy(src_hbm.at[wid, c], sidx)
        pltpu.sync_copy(dst_hbm.at[wid, c], didx)
        gcp = pltpu.async_copy(xw_hbm.at[sidx], g_buf, sem)
        pltpu.sync_copy(t_hbm.at[pl.ds(ebase, CHUNK)], t_buf)
        gcp.wait()

        @pl.loop(0, CHUNK)
        def _edge(e):
            for j in range(H // 16):
                sl = pl.ds(j * 16, 16)
                v = t_buf[e, sl] + g_buf[e, sl]
                q = v * v * (-_C2) - _C1
                d = jnp.exp(v * q) + 1.0
                t_buf[e, sl] = v / d + c_buf[0, sl]

        pltpu.sync_copy(t_buf, s_sh.at[didx], add=True)

    plsc.subcore_barrier()

    @pl.when(sid < NT_IO)
    def _copy_out():
        for k in range(nfull):
            off = rbase + k * CHUNK
            pltpu.sync_copy(s_sh.at[pl.ds(off, CHUNK)], t_buf)
            pltpu.sync_copy(t_buf, s_out.at[cid, pl.ds(off, CHUNK)])
        off = rbase + nfull * CHUNK
        pltpu.sync_copy(s_sh.at[pl.ds(off, rem)], t_buf.at[pl.ds(0, rem)])
        pltpu.sync_copy(t_buf.at[pl.ds(0, rem)], s_out.at[cid, pl.ds(off, rem)])


def _node_body(s_ref, x_ref, ew2_ref, uw1_ref, ub1_ref, uw2_ref, ub2_ref,
               eps_ref, o_ref):
    s = s_ref[0] + s_ref[1]
    agg = jnp.dot(s, ew2_ref[...], preferred_element_type=jnp.float32,
                  precision=lax.Precision.HIGHEST)
    y = (1.0 + eps_ref[0]) * x_ref[...] + agg
    h = jax.nn.gelu(
        jnp.dot(y, uw1_ref[...], preferred_element_type=jnp.float32,
                precision=lax.Precision.HIGHEST)
        + ub1_ref[...]
    )
    o_ref[...] = (
        jnp.dot(h, uw2_ref[...], preferred_element_type=jnp.float32,
                precision=lax.Precision.HIGHEST)
        + ub2_ref[...]
    )


def _node_stage(s2, x, eW2, uW1, ub1, uW2, ub2, eps):
    block = 1000
    grid = N_NODES // block
    return pl.pallas_call(
        _node_body,
        grid=(grid,),
        in_specs=[
            pl.BlockSpec((NC, block, H), lambda i: (0, i, 0)),
            pl.BlockSpec((block, H), lambda i: (i, 0)),
            pl.BlockSpec((H, H), lambda i: (0, 0)),
            pl.BlockSpec((H, H), lambda i: (0, 0)),
            pl.BlockSpec((1, H), lambda i: (0, 0)),
            pl.BlockSpec((H, H), lambda i: (0, 0)),
            pl.BlockSpec((1, H), lambda i: (0, 0)),
            pl.BlockSpec(memory_space=pltpu.SMEM),
        ],
        out_specs=pl.BlockSpec((block, H), lambda i: (i, 0)),
        out_shape=jax.ShapeDtypeStruct((N_NODES, H), jnp.float32),
    )(s2, x, eW2, uW1, ub1, uW2, ub2, eps)


def kernel(x, edge_index, edge_attr, eW1, eb1, eW2, eb2, uW1, ub1, uW2, ub2, eps):
    src = edge_index[0].astype(jnp.int32).reshape(NW, NCHUNK, CHUNK)
    dst = edge_index[1].astype(jnp.int32).reshape(NW, NCHUNK, CHUNK)

    c = jnp.linalg.solve(eW2.T, eb2).reshape(1, H)

    t = _mm_bias(edge_attr, eW1, eb1.reshape(1, H), block_rows=2560)
    xw = _mm_bias(x, eW1, jnp.zeros((1, H), jnp.float32), block_rows=2000)

    s2 = _sc_edge(t, xw, src, dst, c)

    return _node_stage(s2, x, eW2, uW1, ub1.reshape(1, H),
                       uW2, ub2.reshape(1, H), eps)


# speedup vs baseline: 2.6847x; 1.4887x over previous
import functools
import math

import jax
import jax.numpy as jnp
from jax import lax
from jax.experimental import pallas as pl
from jax.experimental.pallas import tpu as pltpu
from jax.experimental.pallas import tpu_sc as plsc

N_NODES = 10000
N_EDGES = 320000
H = 128

NC = 2
NS = 16
NW = NC * NS
EPW = N_EDGES // NW
CHUNK = 80
NCHUNK = EPW // CHUNK
NT_IO = 10
RPT = N_NODES // NT_IO

_C1 = 2.0 * math.sqrt(2.0 / math.pi)
_C2 = _C1 * 0.044715


def _mm_bias_body(a_ref, w_ref, b_ref, o_ref):
    o_ref[...] = (
        jnp.dot(a_ref[...], w_ref[...],
                preferred_element_type=jnp.float32,
                precision=lax.Precision.DEFAULT)
        + b_ref[...]
    )


def _mm_bias(a, w, b, block_rows):
    rows = a.shape[0]
    grid = rows // block_rows
    return pl.pallas_call(
        _mm_bias_body,
        grid=(grid,),
        in_specs=[
            pl.BlockSpec((block_rows, H), lambda i: (i, 0)),
            pl.BlockSpec((H, H), lambda i: (0, 0)),
            pl.BlockSpec((1, H), lambda i: (0, 0)),
        ],
        out_specs=pl.BlockSpec((block_rows, H), lambda i: (i, 0)),
        out_shape=jax.ShapeDtypeStruct((rows, H), jnp.float32),
    )(a, w, b)


@functools.partial(
    pl.kernel,
    out_type=jax.ShapeDtypeStruct((NC, N_NODES, H), jnp.float32),
    mesh=plsc.VectorSubcoreMesh(core_axis_name="c", subcore_axis_name="s"),
    scratch_types=[
        pltpu.VMEM((2, CHUNK, H), jnp.float32),
        pltpu.VMEM((2, CHUNK, H), jnp.float32),
        pltpu.VMEM((1, H), jnp.float32),
        pltpu.VMEM((2, CHUNK), jnp.int32),
        pltpu.VMEM((2, CHUNK), jnp.int32),
        pltpu.VMEM((2, CHUNK), jnp.int32),
        pltpu.SemaphoreType.DMA,
        pltpu.SemaphoreType.DMA,
        pltpu.SemaphoreType.DMA,
        pltpu.SemaphoreType.DMA,
        pltpu.SemaphoreType.DMA,
        pltpu.SemaphoreType.DMA,
        pltpu.SemaphoreType.DMA,
        pltpu.SemaphoreType.DMA,
        pltpu.VMEM_SHARED((N_NODES, H), jnp.float32),
    ],
)
def _sc_edge(t_hbm, xw_hbm, src_hbm, dst_hbm, c_hbm, s_out,
             t_buf2, g_buf2, c_buf, sidx2, didx2, dscat2,
             isem0, isem1, tsem0, tsem1, gsem0, gsem1, ssem0, ssem1, s_sh):
    cid = lax.axis_index("c")
    sid = lax.axis_index("s")
    wid = cid * NS + sid

    zero16 = jnp.zeros((16,), jnp.float32)

    @pl.loop(0, CHUNK)
    def _init(e):
        for j in range(H // 16):
            t_buf2[0, e, pl.ds(j * 16, 16)] = zero16

    pltpu.sync_copy(c_hbm, c_buf)

    rbase = sid * RPT
    nfull = RPT // CHUNK
    rem = RPT - nfull * CHUNK

    @pl.when(sid < NT_IO)
    def _zero():
        for k in range(nfull):
            pltpu.sync_copy(t_buf2.at[0], s_sh.at[pl.ds(rbase + k * CHUNK, CHUNK)])
        pltpu.sync_copy(t_buf2.at[0, pl.ds(0, rem)],
                        s_sh.at[pl.ds(rbase + nfull * CHUNK, rem)])

    plsc.subcore_barrier()

    ebase0 = wid * EPW
    isem = (isem0, isem1)
    tsem = (tsem0, tsem1)
    gsem = (gsem0, gsem1)
    ssem = (ssem0, ssem1)

    def idx_issue(c, p):
        pltpu.async_copy(src_hbm.at[wid, c], sidx2.at[p], isem[p])
        pltpu.async_copy(dst_hbm.at[wid, c], didx2.at[p], isem[p])

    def idx_wait(p):
        pltpu.make_async_copy(src_hbm.at[wid, 0], sidx2.at[p], isem[p]).wait()
        pltpu.make_async_copy(dst_hbm.at[wid, 0], didx2.at[p], isem[p]).wait()

    def in_issue(c, p):
        ebase = ebase0 + c * CHUNK
        pltpu.async_copy(t_hbm.at[pl.ds(ebase, CHUNK)], t_buf2.at[p], tsem[p])
        pltpu.async_copy(xw_hbm.at[sidx2.at[p]], g_buf2.at[p], gsem[p])

    def in_wait(p):
        pltpu.make_async_copy(t_hbm.at[pl.ds(0, CHUNK)], t_buf2.at[p],
                              tsem[p]).wait()
        pltpu.make_async_copy(xw_hbm.at[sidx2.at[p]], g_buf2.at[p],
                              gsem[p]).wait()

    def scat_wait(p):
        pltpu.make_async_copy(t_buf2.at[p], s_sh.at[dscat2.at[p]],
                              ssem[p]).wait()

    idx_issue(0, 0)
    idx_issue(1, 1)
    idx_wait(0)
    in_issue(0, 0)

    @pl.loop(0, (NCHUNK + 1) // 2 + 1)
    def _pipe(g):
        for p in (0, 1):
            c = 2 * g + p
            q = 1 - p

            @pl.when(c < NCHUNK)
            def _work():
                in_wait(p)

                @pl.when(c >= 1)
                def _drain_prev():
                    scat_wait(q)

                @pl.when(c + 1 < NCHUNK)
                def _next_in():
                    idx_wait(q)
                    in_issue(c + 1, q)

                @pl.loop(0, CHUNK)
                def _edge(e):
                    for j in range(H // 16):
                        sl = pl.ds(j * 16, 16)
                        v = t_buf2[p, e, sl] + g_buf2[p, e, sl]
                        qq = v * v * (-_C2) - _C1
                        d = jnp.exp(v * qq) + 1.0
                        t_buf2[p, e, sl] = v / d + c_buf[0, sl]

                for k in range(CHUNK // 16):
                    sl = pl.ds(k * 16, 16)
                    dscat2[p, sl] = didx2[p, sl]

                pltpu.async_copy(t_buf2.at[p], s_sh.at[dscat2.at[p]],
                                 ssem[p], add=True)

                @pl.when(c + 2 < NCHUNK)
                def _next_idx():
                    idx_issue(c + 2, p)

    scat_wait((NCHUNK - 1) % 2)

    plsc.subcore_barrier()

    @pl.when(sid < NT_IO)
    def _copy_out():
        for k in range(nfull):
            off = rbase + k * CHUNK
            pltpu.sync_copy(s_sh.at[pl.ds(off, CHUNK)], t_buf2.at[0])
            pltpu.sync_copy(t_buf2.at[0], s_out.at[cid, pl.ds(off, CHUNK)])
        off = rbase + nfull * CHUNK
        pltpu.sync_copy(s_sh.at[pl.ds(off, rem)], t_buf2.at[0, pl.ds(0, rem)])
        pltpu.sync_copy(t_buf2.at[0, pl.ds(0, rem)],
                        s_out.at[cid, pl.ds(off, rem)])


def _node_body(s_ref, x_ref, ew2_ref, uw1_ref, ub1_ref, uw2_ref, ub2_ref,
               eps_ref, o_ref):
    s = s_ref[0] + s_ref[1]
    agg = jnp.dot(s, ew2_ref[...], preferred_element_type=jnp.float32,
                  precision=lax.Precision.DEFAULT)
    y = (1.0 + eps_ref[0]) * x_ref[...] + agg
    h = jax.nn.gelu(
        jnp.dot(y, uw1_ref[...], preferred_element_type=jnp.float32,
                precision=lax.Precision.DEFAULT)
        + ub1_ref[...]
    )
    o_ref[...] = (
        jnp.dot(h, uw2_ref[...], preferred_element_type=jnp.float32,
                precision=lax.Precision.DEFAULT)
        + ub2_ref[...]
    )


def _node_stage(s2, x, eW2, uW1, ub1, uW2, ub2, eps):
    block = 1000
    grid = N_NODES // block
    return pl.pallas_call(
        _node_body,
        grid=(grid,),
        in_specs=[
            pl.BlockSpec((NC, block, H), lambda i: (0, i, 0)),
            pl.BlockSpec((block, H), lambda i: (i, 0)),
            pl.BlockSpec((H, H), lambda i: (0, 0)),
            pl.BlockSpec((H, H), lambda i: (0, 0)),
            pl.BlockSpec((1, H), lambda i: (0, 0)),
            pl.BlockSpec((H, H), lambda i: (0, 0)),
            pl.BlockSpec((1, H), lambda i: (0, 0)),
            pl.BlockSpec(memory_space=pltpu.SMEM),
        ],
        out_specs=pl.BlockSpec((block, H), lambda i: (i, 0)),
        out_shape=jax.ShapeDtypeStruct((N_NODES, H), jnp.float32),
    )(s2, x, eW2, uW1, ub1, uW2, ub2, eps)


def kernel(x, edge_index, edge_attr, eW1, eb1, eW2, eb2, uW1, ub1, uW2, ub2, eps):
    src = edge_index[0].astype(jnp.int32).reshape(NW, NCHUNK, CHUNK)
    dst = edge_index[1].astype(jnp.int32).reshape(NW, NCHUNK, CHUNK)

    c = jnp.linalg.solve(eW2.T, eb2).reshape(1, H)

    t = _mm_bias(edge_attr, eW1, eb1.reshape(1, H), block_rows=2560)
    xw = _mm_bias(x, eW1, jnp.zeros((1, H), jnp.float32), block_rows=2000)

    s2 = _sc_edge(t, xw, src, dst, c)

    return _node_stage(s2, x, eW2, uW1, ub1.reshape(1, H),
                       uW2, ub2.reshape(1, H), eps)
